# baseline (speedup 1.0000x reference)
import functools

import jax
import jax.numpy as jnp
from jax import lax
from jax.experimental import pallas as pl
from jax.experimental.pallas import tpu as pltpu
from jax.experimental.pallas import tpu_sc as plsc

_B = 16384
_D_MLP = 32
_D_MF = 16
_NC = 2
_NS = 16
_NW = _NC * _NS
_BPW = _B // _NW
_CH = 128
_NCH = _BPW // _CH


def _gather_body(u_hbm, i_hbm, t_um, t_im, t_umf, t_imf,
                 o_um, o_im, o_umf, o_imf,
                 uidx, iidx, b_um, b_im, b_umf, b_imf, sem):
    wid = lax.axis_index("s") * _NC + lax.axis_index("c")
    base = wid * _BPW
    for j in range(_NCH):
        pltpu.sync_copy(u_hbm.at[pl.ds(base + j * _CH, _CH)], uidx.at[j])
        pltpu.sync_copy(i_hbm.at[pl.ds(base + j * _CH, _CH)], iidx.at[j])
    cps = []
    for j in range(_NCH):
        sl = pl.ds(j * _CH, _CH)
        cps.append(pltpu.async_copy(t_um.at[uidx.at[j]], b_um.at[sl], sem))
        cps.append(pltpu.async_copy(t_im.at[iidx.at[j]], b_im.at[sl], sem))
        cps.append(pltpu.async_copy(t_umf.at[uidx.at[j]], b_umf.at[sl], sem))
        cps.append(pltpu.async_copy(t_imf.at[iidx.at[j]], b_imf.at[sl], sem))
    for cp in cps:
        cp.wait()
    out_sl = pl.ds(base, _BPW)
    pltpu.sync_copy(b_um, o_um.at[out_sl])
    pltpu.sync_copy(b_im, o_im.at[out_sl])
    pltpu.sync_copy(b_umf, o_umf.at[out_sl])
    pltpu.sync_copy(b_imf, o_imf.at[out_sl])


@functools.lru_cache(maxsize=None)
def _make_gather():
  return functools.partial(
    pl.kernel,
    mesh=plsc.VectorSubcoreMesh(core_axis_name="c", subcore_axis_name="s"),
    compiler_params=pltpu.CompilerParams(use_tc_tiling_on_sc=False),
    out_type=[
        jax.ShapeDtypeStruct((_B, _D_MLP), jnp.float32),
        jax.ShapeDtypeStruct((_B, _D_MLP), jnp.float32),
        jax.ShapeDtypeStruct((_B, _D_MF), jnp.float32),
        jax.ShapeDtypeStruct((_B, _D_MF), jnp.float32),
    ],
    scratch_types=[
        pltpu.VMEM((_NCH, _CH), jnp.int32),
        pltpu.VMEM((_NCH, _CH), jnp.int32),
        pltpu.VMEM((_BPW, _D_MLP), jnp.float32),
        pltpu.VMEM((_BPW, _D_MLP), jnp.float32),
        pltpu.VMEM((_BPW, _D_MF), jnp.float32),
        pltpu.VMEM((_BPW, _D_MF), jnp.float32),
        pltpu.SemaphoreType.DMA,
    ],
  )(_gather_body)


def _mlp_body(um_ref, im_ref, umf_ref, imf_ref,
              w1u_ref, w1i_ref, b1_ref, w2_ref, b2_ref, w3_ref, b3_ref,
              wom_ref, wof_ref, bo_ref, out_ref):
    x = jnp.dot(um_ref[...], w1u_ref[...], preferred_element_type=jnp.float32)
    x = x + jnp.dot(im_ref[...], w1i_ref[...], preferred_element_type=jnp.float32)
    h = jnp.maximum(x + b1_ref[...], 0.0)
    h = jnp.maximum(
        jnp.dot(h, w2_ref[...], preferred_element_type=jnp.float32) + b2_ref[...], 0.0)
    h = jnp.maximum(
        jnp.dot(h, w3_ref[...], preferred_element_type=jnp.float32) + b3_ref[...], 0.0)
    mf = umf_ref[...] * imf_ref[...]
    logit = (jnp.dot(h, wom_ref[...], preferred_element_type=jnp.float32)
             + jnp.dot(mf, wof_ref[...], preferred_element_type=jnp.float32)
             + bo_ref[...])
    out_ref[...] = 1.0 / (1.0 + jnp.exp(-logit))


_BLK = 2048


def _mlp(um, im, umf, imf, w1u, w1i, b1, w2, b2, w3, b3, wom, wof, bo):
    full = lambda n: (0, 0)
    row = lambda n: (n, 0)
    return pl.pallas_call(
        _mlp_body,
        grid=(_B // _BLK,),
        in_specs=[
            pl.BlockSpec((_BLK, _D_MLP), row),
            pl.BlockSpec((_BLK, _D_MLP), row),
            pl.BlockSpec((_BLK, _D_MF), row),
            pl.BlockSpec((_BLK, _D_MF), row),
            pl.BlockSpec(w1u.shape, full),
            pl.BlockSpec(w1i.shape, full),
            pl.BlockSpec(b1.shape, full),
            pl.BlockSpec(w2.shape, full),
            pl.BlockSpec(b2.shape, full),
            pl.BlockSpec(w3.shape, full),
            pl.BlockSpec(b3.shape, full),
            pl.BlockSpec(wom.shape, full),
            pl.BlockSpec(wof.shape, full),
            pl.BlockSpec(bo.shape, full),
        ],
        out_specs=pl.BlockSpec((_BLK, 1), row),
        out_shape=jax.ShapeDtypeStruct((_B, 1), jnp.float32),
    )(um, im, umf, imf, w1u, w1i, b1, w2, b2, w3, b3, wom, wof, bo)


def kernel(u, i, emb_user_mlp, emb_item_mlp, emb_user_mf, emb_item_mf,
           W1, b1, W2, b2, W3, b3, W_out, b_out):
    u = u.astype(jnp.int32)
    i = i.astype(jnp.int32)
    um, im, umf, imf = _make_gather()(u, i, emb_user_mlp, emb_item_mlp,
                                      emb_user_mf, emb_item_mf)
    w1u = W1[:_D_MLP]
    w1i = W1[_D_MLP:]
    wom = W_out[:-_D_MF]
    wof = W_out[-_D_MF:]
    y = _mlp(um, im, umf, imf,
             w1u, w1i, b1.reshape(1, -1), W2, b2.reshape(1, -1),
             W3, b3.reshape(1, -1), wom, wof, b_out.reshape(1, 1))
    return y


# speedup vs baseline: 1.0298x; 1.0298x over previous
import functools

import jax
import jax.numpy as jnp
from jax import lax
from jax.experimental import pallas as pl
from jax.experimental.pallas import tpu as pltpu
from jax.experimental.pallas import tpu_sc as plsc

_B = 16384
_V = 1_000_000
_D_MLP = 32
_D_MF = 16
_NC = 2
_NS = 16
_NW = _NC * _NS
_BPW = _B // _NW
_CH = 128
_NCH = _BPW // _CH

_COLS = 2048
_GRID_A = (_V + _COLS - 1) // _COLS


def _relayout_body(tu_ref, ti_ref, fu_ref, fi_ref, pu_ref, pi_ref, qu_ref, qi_ref):
    def pack(x_ref, planes, d):
        per = 128 // d
        x2 = x_ref[...].reshape(planes * 8, _COLS)
        m = x2.T
        m3 = m.reshape(_COLS // per, per, d)
        return jnp.concatenate([m3[:, k] for k in range(per)], axis=1)

    pu_ref[...] = pack(tu_ref, 4, _D_MLP)
    pi_ref[...] = pack(ti_ref, 4, _D_MLP)
    qu_ref[...] = pack(fu_ref, 2, _D_MF)
    qi_ref[...] = pack(fi_ref, 2, _D_MF)


def _relayout(tu, ti, fu, fi):
    blk3 = lambda p: pl.BlockSpec((p, 8, _COLS), lambda n: (0, 0, n))
    return pl.pallas_call(
        _relayout_body,
        grid=(_GRID_A,),
        in_specs=[blk3(4), blk3(4), blk3(2), blk3(2)],
        out_specs=[
            pl.BlockSpec((_COLS // 4, 128), lambda n: (n, 0)),
            pl.BlockSpec((_COLS // 4, 128), lambda n: (n, 0)),
            pl.BlockSpec((_COLS // 8, 128), lambda n: (n, 0)),
            pl.BlockSpec((_COLS // 8, 128), lambda n: (n, 0)),
        ],
        out_shape=[
            jax.ShapeDtypeStruct((_V // 4, 128), jnp.float32),
            jax.ShapeDtypeStruct((_V // 4, 128), jnp.float32),
            jax.ShapeDtypeStruct((_V // 8, 128), jnp.float32),
            jax.ShapeDtypeStruct((_V // 8, 128), jnp.float32),
        ],
    )(tu, ti, fu, fi)


def _gather_body(u2_hbm, i2_hbm, u3_hbm, i3_hbm, pu, pi, qu, qi,
                 o_um, o_im, o_umf, o_imf,
                 u2x, i2x, u3x, i3x, b_um, b_im, b_umf, b_imf, sem):
    wid = lax.axis_index("s") * _NC + lax.axis_index("c")
    base = wid * _BPW
    for j in range(_NCH):
        sl = pl.ds(base + j * _CH, _CH)
        pltpu.sync_copy(u2_hbm.at[sl], u2x.at[j])
        pltpu.sync_copy(i2_hbm.at[sl], i2x.at[j])
        pltpu.sync_copy(u3_hbm.at[sl], u3x.at[j])
        pltpu.sync_copy(i3_hbm.at[sl], i3x.at[j])
    for j in range(_NCH):
        cps = [
            pltpu.async_copy(pu.at[u2x.at[j]], b_um, sem),
            pltpu.async_copy(pi.at[i2x.at[j]], b_im, sem),
            pltpu.async_copy(qu.at[u3x.at[j]], b_umf, sem),
            pltpu.async_copy(qi.at[i3x.at[j]], b_imf, sem),
        ]
        for cp in cps:
            cp.wait()
        sl = pl.ds(j * _CH, _CH)
        pltpu.sync_copy(b_um, o_um.at[wid].at[sl])
        pltpu.sync_copy(b_im, o_im.at[wid].at[sl])
        pltpu.sync_copy(b_umf, o_umf.at[wid].at[sl])
        pltpu.sync_copy(b_imf, o_imf.at[wid].at[sl])


@functools.lru_cache(maxsize=None)
def _make_gather():
  return functools.partial(
    pl.kernel,
    mesh=plsc.VectorSubcoreMesh(core_axis_name="c", subcore_axis_name="s"),
    out_type=[
        jax.ShapeDtypeStruct((_NW, _BPW, 128), jnp.float32),
        jax.ShapeDtypeStruct((_NW, _BPW, 128), jnp.float32),
        jax.ShapeDtypeStruct((_NW, _BPW, 128), jnp.float32),
        jax.ShapeDtypeStruct((_NW, _BPW, 128), jnp.float32),
    ],
    scratch_types=[
        pltpu.VMEM((_NCH, _CH), jnp.int32),
        pltpu.VMEM((_NCH, _CH), jnp.int32),
        pltpu.VMEM((_NCH, _CH), jnp.int32),
        pltpu.VMEM((_NCH, _CH), jnp.int32),
        pltpu.VMEM((_CH, 128), jnp.float32),
        pltpu.VMEM((_CH, 128), jnp.float32),
        pltpu.VMEM((_CH, 128), jnp.float32),
        pltpu.VMEM((_CH, 128), jnp.float32),
        pltpu.SemaphoreType.DMA,
    ],
  )(_gather_body)


def _mlp_body(u_ref, i_ref, um_ref, im_ref, umf_ref, imf_ref,
              w1u_ref, w1i_ref, b1_ref, w2_ref, b2_ref, w3_ref, b3_ref,
              wom_ref, wof_ref, bo_ref, out_ref):
    u = u_ref[...]
    i = i_ref[...]

    def select(pk, phase, d):
        acc = jnp.zeros((_BPW, d), jnp.float32)
        for j in range(128 // d):
            acc = acc + jnp.where(phase == j, pk[:, j * d:(j + 1) * d], 0.0)
        return acc

    xu = select(um_ref[0], u & 3, _D_MLP)
    xi = select(im_ref[0], i & 3, _D_MLP)
    mu = select(umf_ref[0], u & 7, _D_MF)
    mi = select(imf_ref[0], i & 7, _D_MF)
    x = jnp.dot(xu, w1u_ref[...], preferred_element_type=jnp.float32)
    x = x + jnp.dot(xi, w1i_ref[...], preferred_element_type=jnp.float32)
    h = jnp.maximum(x + b1_ref[...], 0.0)
    h = jnp.maximum(
        jnp.dot(h, w2_ref[...], preferred_element_type=jnp.float32) + b2_ref[...], 0.0)
    h = jnp.maximum(
        jnp.dot(h, w3_ref[...], preferred_element_type=jnp.float32) + b3_ref[...], 0.0)
    mf = mu * mi
    logit = (jnp.dot(h, wom_ref[...], preferred_element_type=jnp.float32)
             + jnp.dot(mf, wof_ref[...], preferred_element_type=jnp.float32)
             + bo_ref[...])
    out_ref[...] = 1.0 / (1.0 + jnp.exp(-logit))


def _mlp(u, i, um, im, umf, imf, w1u, w1i, b1, w2, b2, w3, b3, wom, wof, bo):
    full = lambda n: (0, 0)
    row2 = lambda n: (n, 0)
    blk3 = lambda n: (n, 0, 0)
    return pl.pallas_call(
        _mlp_body,
        grid=(_NW,),
        in_specs=[
            pl.BlockSpec((_BPW, 1), row2),
            pl.BlockSpec((_BPW, 1), row2),
            pl.BlockSpec((1, _BPW, 128), blk3),
            pl.BlockSpec((1, _BPW, 128), blk3),
            pl.BlockSpec((1, _BPW, 128), blk3),
            pl.BlockSpec((1, _BPW, 128), blk3),
            pl.BlockSpec(w1u.shape, full),
            pl.BlockSpec(w1i.shape, full),
            pl.BlockSpec(b1.shape, full),
            pl.BlockSpec(w2.shape, full),
            pl.BlockSpec(b2.shape, full),
            pl.BlockSpec(w3.shape, full),
            pl.BlockSpec(b3.shape, full),
            pl.BlockSpec(wom.shape, full),
            pl.BlockSpec(wof.shape, full),
            pl.BlockSpec(bo.shape, full),
        ],
        out_specs=pl.BlockSpec((_BPW, 1), row2),
        out_shape=jax.ShapeDtypeStruct((_B, 1), jnp.float32),
    )(u, i, um, im, umf, imf, w1u, w1i, b1, w2, b2, w3, b3, wom, wof, bo)


def kernel(u, i, emb_user_mlp, emb_item_mlp, emb_user_mf, emb_item_mf,
           W1, b1, W2, b2, W3, b3, W_out, b_out):
    u = u.astype(jnp.int32)
    i = i.astype(jnp.int32)
    pu, pi, qu, qi = _relayout(
        emb_user_mlp.T.reshape(4, 8, _V), emb_item_mlp.T.reshape(4, 8, _V),
        emb_user_mf.T.reshape(2, 8, _V), emb_item_mf.T.reshape(2, 8, _V))
    um, im, umf, imf = _make_gather()(
        u >> 2, i >> 2, u >> 3, i >> 3, pu, pi, qu, qi)
    y = _mlp(u.reshape(_B, 1), i.reshape(_B, 1), um, im, umf, imf,
             W1[:_D_MLP], W1[_D_MLP:], b1.reshape(1, -1),
             W2, b2.reshape(1, -1), W3, b3.reshape(1, -1),
             W_out[:-_D_MF], W_out[-_D_MF:], b_out.reshape(1, 1))
    return y


# speedup vs baseline: 1.4261x; 1.3848x over previous
import functools

import jax
import jax.numpy as jnp
from jax import lax
from jax.experimental import pallas as pl
from jax.experimental.pallas import tpu as pltpu
from jax.experimental.pallas import tpu_sc as plsc

_B = 16384
_V = 1_000_000
_D_MLP = 32
_D_MF = 16
_NC = 2
_NS = 16
_NW = _NC * _NS
_BPW = _B // _NW
_CH = 128
_NCH = _BPW // _CH

_COLS = 4096
_GRID_A = (_V + _COLS - 1) // _COLS


def _relayout_body(tu_ref, ti_ref, fu_ref, fi_ref, r32_ref, m32_ref,
                   r16_ref, m16_ref, pu_ref, pi_ref, qu_ref, qi_ref):
    def pack(x_ref, planes, d, rep_ref, msk_ref):
        per = 128 // d
        x2 = x_ref[...].reshape(planes * 8, _COLS)
        m = lax.dot_general(x2, rep_ref[...], (((0,), (0,)), ((), ())),
                            preferred_element_type=jnp.float32)
        m4 = m.reshape(_COLS // per, per, 128)
        return jnp.sum(m4 * msk_ref[...][None], axis=1)

    pu_ref[...] = pack(tu_ref, 4, _D_MLP, r32_ref, m32_ref)
    pi_ref[...] = pack(ti_ref, 4, _D_MLP, r32_ref, m32_ref)
    qu_ref[...] = pack(fu_ref, 2, _D_MF, r16_ref, m16_ref)
    qi_ref[...] = pack(fi_ref, 2, _D_MF, r16_ref, m16_ref)


def _relayout(tu, ti, fu, fi):
    blk3 = lambda p: pl.BlockSpec((p, 8, _COLS), lambda n: (0, 0, n))
    full = lambda n: (0, 0)
    return pl.pallas_call(
        _relayout_body,
        grid=(_GRID_A,),
        in_specs=[blk3(4), blk3(4), blk3(2), blk3(2),
                  pl.BlockSpec((_D_MLP, 128), full),
                  pl.BlockSpec((4, 128), full),
                  pl.BlockSpec((_D_MF, 128), full),
                  pl.BlockSpec((8, 128), full)],
        out_specs=[
            pl.BlockSpec((_COLS // 4, 128), lambda n: (n, 0)),
            pl.BlockSpec((_COLS // 4, 128), lambda n: (n, 0)),
            pl.BlockSpec((_COLS // 8, 128), lambda n: (n, 0)),
            pl.BlockSpec((_COLS // 8, 128), lambda n: (n, 0)),
        ],
        out_shape=[
            jax.ShapeDtypeStruct((_V // 4, 128), jnp.float32),
            jax.ShapeDtypeStruct((_V // 4, 128), jnp.float32),
            jax.ShapeDtypeStruct((_V // 8, 128), jnp.float32),
            jax.ShapeDtypeStruct((_V // 8, 128), jnp.float32),
        ],
    )(tu, ti, fu, fi, _rep(_D_MLP), _msk(_D_MLP), _rep(_D_MF), _msk(_D_MF))


def _rep(d):
    k = jnp.arange(128)
    f = jnp.arange(d).reshape(-1, 1)
    return (k % d == f).astype(jnp.float32)


def _msk(d):
    per = 128 // d
    k = jnp.arange(128)
    j = jnp.arange(per).reshape(-1, 1)
    return (k // d == j).astype(jnp.float32)


def _gather_body(u2_hbm, i2_hbm, u3_hbm, i3_hbm, pu, pi, qu, qi,
                 o_um, o_im, o_umf, o_imf,
                 u2x, i2x, u3x, i3x, b_um, b_im, b_umf, b_imf, sem):
    wid = lax.axis_index("s") * _NC + lax.axis_index("c")
    base = wid * _BPW
    for j in range(_NCH):
        sl = pl.ds(base + j * _CH, _CH)
        pltpu.sync_copy(u2_hbm.at[sl], u2x.at[j])
        pltpu.sync_copy(i2_hbm.at[sl], i2x.at[j])
        pltpu.sync_copy(u3_hbm.at[sl], u3x.at[j])
        pltpu.sync_copy(i3_hbm.at[sl], i3x.at[j])
    for j in range(_NCH):
        cps = [
            pltpu.async_copy(pu.at[u2x.at[j]], b_um, sem),
            pltpu.async_copy(pi.at[i2x.at[j]], b_im, sem),
            pltpu.async_copy(qu.at[u3x.at[j]], b_umf, sem),
            pltpu.async_copy(qi.at[i3x.at[j]], b_imf, sem),
        ]
        for cp in cps:
            cp.wait()
        sl = pl.ds(j * _CH, _CH)
        pltpu.sync_copy(b_um, o_um.at[wid].at[sl])
        pltpu.sync_copy(b_im, o_im.at[wid].at[sl])
        pltpu.sync_copy(b_umf, o_umf.at[wid].at[sl])
        pltpu.sync_copy(b_imf, o_imf.at[wid].at[sl])


@functools.lru_cache(maxsize=None)
def _make_gather():
  return functools.partial(
    pl.kernel,
    mesh=plsc.VectorSubcoreMesh(core_axis_name="c", subcore_axis_name="s"),
    out_type=[
        jax.ShapeDtypeStruct((_NW, _BPW, 128), jnp.float32),
        jax.ShapeDtypeStruct((_NW, _BPW, 128), jnp.float32),
        jax.ShapeDtypeStruct((_NW, _BPW, 128), jnp.float32),
        jax.ShapeDtypeStruct((_NW, _BPW, 128), jnp.float32),
    ],
    scratch_types=[
        pltpu.VMEM((_NCH, _CH), jnp.int32),
        pltpu.VMEM((_NCH, _CH), jnp.int32),
        pltpu.VMEM((_NCH, _CH), jnp.int32),
        pltpu.VMEM((_NCH, _CH), jnp.int32),
        pltpu.VMEM((_CH, 128), jnp.float32),
        pltpu.VMEM((_CH, 128), jnp.float32),
        pltpu.VMEM((_CH, 128), jnp.float32),
        pltpu.VMEM((_CH, 128), jnp.float32),
        pltpu.SemaphoreType.DMA,
    ],
  )(_gather_body)


def _mlp_body(u_ref, i_ref, um_ref, im_ref, umf_ref, imf_ref,
              w1u_ref, w1i_ref, b1_ref, w2_ref, b2_ref, w3_ref, b3_ref,
              wom_ref, wof_ref, bo_ref, out_ref):
    u = u_ref[...]
    i = i_ref[...]

    def select(pk, phase, d):
        acc = jnp.zeros((_BPW, d), jnp.float32)
        for j in range(128 // d):
            acc = acc + jnp.where(phase == j, pk[:, j * d:(j + 1) * d], 0.0)
        return acc

    xu = select(um_ref[0], u & 3, _D_MLP)
    xi = select(im_ref[0], i & 3, _D_MLP)
    mu = select(umf_ref[0], u & 7, _D_MF)
    mi = select(imf_ref[0], i & 7, _D_MF)
    x = jnp.dot(xu, w1u_ref[...], preferred_element_type=jnp.float32)
    x = x + jnp.dot(xi, w1i_ref[...], preferred_element_type=jnp.float32)
    h = jnp.maximum(x + b1_ref[...], 0.0)
    h = jnp.maximum(
        jnp.dot(h, w2_ref[...], preferred_element_type=jnp.float32) + b2_ref[...], 0.0)
    h = jnp.maximum(
        jnp.dot(h, w3_ref[...], preferred_element_type=jnp.float32) + b3_ref[...], 0.0)
    mf = mu * mi
    logit = (jnp.dot(h, wom_ref[...], preferred_element_type=jnp.float32)
             + jnp.dot(mf, wof_ref[...], preferred_element_type=jnp.float32)
             + bo_ref[...])
    out_ref[...] = 1.0 / (1.0 + jnp.exp(-logit))


def _mlp(u, i, um, im, umf, imf, w1u, w1i, b1, w2, b2, w3, b3, wom, wof, bo):
    full = lambda n: (0, 0)
    row2 = lambda n: (n, 0)
    blk3 = lambda n: (n, 0, 0)
    return pl.pallas_call(
        _mlp_body,
        grid=(_NW,),
        in_specs=[
            pl.BlockSpec((_BPW, 1), row2),
            pl.BlockSpec((_BPW, 1), row2),
            pl.BlockSpec((1, _BPW, 128), blk3),
            pl.BlockSpec((1, _BPW, 128), blk3),
            pl.BlockSpec((1, _BPW, 128), blk3),
            pl.BlockSpec((1, _BPW, 128), blk3),
            pl.BlockSpec(w1u.shape, full),
            pl.BlockSpec(w1i.shape, full),
            pl.BlockSpec(b1.shape, full),
            pl.BlockSpec(w2.shape, full),
            pl.BlockSpec(b2.shape, full),
            pl.BlockSpec(w3.shape, full),
            pl.BlockSpec(b3.shape, full),
            pl.BlockSpec(wom.shape, full),
            pl.BlockSpec(wof.shape, full),
            pl.BlockSpec(bo.shape, full),
        ],
        out_specs=pl.BlockSpec((_BPW, 1), row2),
        out_shape=jax.ShapeDtypeStruct((_B, 1), jnp.float32),
    )(u, i, um, im, umf, imf, w1u, w1i, b1, w2, b2, w3, b3, wom, wof, bo)


def kernel(u, i, emb_user_mlp, emb_item_mlp, emb_user_mf, emb_item_mf,
           W1, b1, W2, b2, W3, b3, W_out, b_out):
    u = u.astype(jnp.int32)
    i = i.astype(jnp.int32)
    pu, pi, qu, qi = _relayout(
        emb_user_mlp.T.reshape(4, 8, _V), emb_item_mlp.T.reshape(4, 8, _V),
        emb_user_mf.T.reshape(2, 8, _V), emb_item_mf.T.reshape(2, 8, _V))
    um, im, umf, imf = _make_gather()(
        u >> 2, i >> 2, u >> 3, i >> 3, pu, pi, qu, qi)
    y = _mlp(u.reshape(_B, 1), i.reshape(_B, 1), um, im, umf, imf,
             W1[:_D_MLP], W1[_D_MLP:], b1.reshape(1, -1),
             W2, b2.reshape(1, -1), W3, b3.reshape(1, -1),
             W_out[:-_D_MF], W_out[-_D_MF:], b_out.reshape(1, 1))
    return y


# speedup vs baseline: 3.6052x; 2.5279x over previous
import functools

import jax
import jax.numpy as jnp
from jax import lax
from jax.experimental import pallas as pl
from jax.experimental.pallas import tpu as pltpu
from jax.experimental.pallas import tpu_sc as plsc

_B = 16384
_V = 1_000_000
_D_MLP = 32
_D_MF = 16
_F = 2 * _D_MLP + 2 * _D_MF
_NC = 2
_NS = 16
_NW = _NC * _NS
_BPW = _B // _NW
_CH = 128
_NCH = _BPW // _CH

_COLS = 4096
_GRID_A = (_V + _COLS - 1) // _COLS


def _relayout_body(tu_ref, ti_ref, fu_ref, fi_ref, rep_ref, p_ref):
    x2 = jnp.concatenate(
        [tu_ref[...].reshape(_D_MLP, _COLS),
         ti_ref[...].reshape(_D_MLP, _COLS),
         fu_ref[...].reshape(_D_MF, _COLS),
         fi_ref[...].reshape(_D_MF, _COLS)], axis=0)
    p_ref[...] = lax.dot_general(x2, rep_ref[...], (((0,), (0,)), ((), ())),
                                 preferred_element_type=jnp.float32)


def _relayout(tu, ti, fu, fi):
    blk3 = lambda p: pl.BlockSpec((p, 8, _COLS), lambda n: (0, 0, n))
    rep = jnp.eye(_F, 128, dtype=jnp.float32)
    return pl.pallas_call(
        _relayout_body,
        grid=(_GRID_A,),
        in_specs=[blk3(4), blk3(4), blk3(2), blk3(2),
                  pl.BlockSpec((_F, 128), lambda n: (0, 0))],
        out_specs=pl.BlockSpec((_COLS, 128), lambda n: (n, 0)),
        out_shape=jax.ShapeDtypeStruct((_V, 128), jnp.float32),
    )(tu, ti, fu, fi, rep)


def _gather_body(u_hbm, i_hbm, t_p,
                 o_u, o_i,
                 ux, ix, b_u, b_i, sem):
    wid = lax.axis_index("s") * _NC + lax.axis_index("c")
    base = wid * _BPW
    for j in range(_NCH):
        sl = pl.ds(base + j * _CH, _CH)
        pltpu.sync_copy(u_hbm.at[sl], ux.at[j])
        pltpu.sync_copy(i_hbm.at[sl], ix.at[j])
    for j in range(_NCH):
        cps = [
            pltpu.async_copy(t_p.at[ux.at[j]], b_u, sem),
            pltpu.async_copy(t_p.at[ix.at[j]], b_i, sem),
        ]
        for cp in cps:
            cp.wait()
        sl = pl.ds(j * _CH, _CH)
        pltpu.sync_copy(b_u, o_u.at[wid].at[sl])
        pltpu.sync_copy(b_i, o_i.at[wid].at[sl])


@functools.lru_cache(maxsize=None)
def _make_gather():
  return functools.partial(
    pl.kernel,
    mesh=plsc.VectorSubcoreMesh(core_axis_name="c", subcore_axis_name="s"),
    out_type=[
        jax.ShapeDtypeStruct((_NW, _BPW, 128), jnp.float32),
        jax.ShapeDtypeStruct((_NW, _BPW, 128), jnp.float32),
    ],
    scratch_types=[
        pltpu.VMEM((_NCH, _CH), jnp.int32),
        pltpu.VMEM((_NCH, _CH), jnp.int32),
        pltpu.VMEM((_CH, 128), jnp.float32),
        pltpu.VMEM((_CH, 128), jnp.float32),
        pltpu.SemaphoreType.DMA,
    ],
  )(_gather_body)


def _mlp_body(um_ref, im_ref,
              w1u_ref, w1i_ref, b1_ref, w2_ref, b2_ref, w3_ref, b3_ref,
              wom_ref, wof_ref, bo_ref, out_ref):
    pu = um_ref[0]
    pi = im_ref[0]
    xu = pu[:, :_D_MLP]
    xi = pi[:, _D_MLP:2 * _D_MLP]
    mu = pu[:, 2 * _D_MLP:2 * _D_MLP + _D_MF]
    mi = pi[:, 2 * _D_MLP + _D_MF:_F]
    x = jnp.dot(xu, w1u_ref[...], preferred_element_type=jnp.float32)
    x = x + jnp.dot(xi, w1i_ref[...], preferred_element_type=jnp.float32)
    h = jnp.maximum(x + b1_ref[...], 0.0)
    h = jnp.maximum(
        jnp.dot(h, w2_ref[...], preferred_element_type=jnp.float32) + b2_ref[...], 0.0)
    h = jnp.maximum(
        jnp.dot(h, w3_ref[...], preferred_element_type=jnp.float32) + b3_ref[...], 0.0)
    mf = mu * mi
    logit = (jnp.dot(h, wom_ref[...], preferred_element_type=jnp.float32)
             + jnp.dot(mf, wof_ref[...], preferred_element_type=jnp.float32)
             + bo_ref[...])
    out_ref[...] = 1.0 / (1.0 + jnp.exp(-logit))


def _mlp(um, im, w1u, w1i, b1, w2, b2, w3, b3, wom, wof, bo):
    full = lambda n: (0, 0)
    row2 = lambda n: (n, 0)
    blk3 = lambda n: (n, 0, 0)
    return pl.pallas_call(
        _mlp_body,
        grid=(_NW,),
        in_specs=[
            pl.BlockSpec((1, _BPW, 128), blk3),
            pl.BlockSpec((1, _BPW, 128), blk3),
            pl.BlockSpec(w1u.shape, full),
            pl.BlockSpec(w1i.shape, full),
            pl.BlockSpec(b1.shape, full),
            pl.BlockSpec(w2.shape, full),
            pl.BlockSpec(b2.shape, full),
            pl.BlockSpec(w3.shape, full),
            pl.BlockSpec(b3.shape, full),
            pl.BlockSpec(wom.shape, full),
            pl.BlockSpec(wof.shape, full),
            pl.BlockSpec(bo.shape, full),
        ],
        out_specs=pl.BlockSpec((_BPW, 1), row2),
        out_shape=jax.ShapeDtypeStruct((_B, 1), jnp.float32),
    )(um, im, w1u, w1i, b1, w2, b2, w3, b3, wom, wof, bo)


def kernel(u, i, emb_user_mlp, emb_item_mlp, emb_user_mf, emb_item_mf,
           W1, b1, W2, b2, W3, b3, W_out, b_out):
    u = u.astype(jnp.int32)
    i = i.astype(jnp.int32)
    p_cat = _relayout(
        emb_user_mlp.T.reshape(4, 8, _V), emb_item_mlp.T.reshape(4, 8, _V),
        emb_user_mf.T.reshape(2, 8, _V), emb_item_mf.T.reshape(2, 8, _V))
    um, im = _make_gather()(u, i, p_cat)
    y = _mlp(um, im,
             W1[:_D_MLP], W1[_D_MLP:], b1.reshape(1, -1),
             W2, b2.reshape(1, -1), W3, b3.reshape(1, -1),
             W_out[:-_D_MF], W_out[-_D_MF:], b_out.reshape(1, 1))
    return y


# speedup vs baseline: 4.3340x; 1.2021x over previous
import functools

import jax
import jax.numpy as jnp
from jax import lax
from jax.experimental import pallas as pl
from jax.experimental.pallas import tpu as pltpu
from jax.experimental.pallas import tpu_sc as plsc

_B = 16384
_V = 1_000_000
_D_MLP = 32
_D_MF = 16
_F = 2 * _D_MLP + 2 * _D_MF
_NC = 2
_NS = 16
_NW = _NC * _NS
_BPW = _B // _NW
_CH = 128
_NCH = _BPW // _CH

_COLS = 8192
_GRID_A = (_V + _COLS - 1) // _COLS


def _relayout_body(tu_ref, ti_ref, fu_ref, fi_ref, rep_ref, p_ref):
    x2 = jnp.concatenate(
        [tu_ref[...].reshape(_D_MLP, _COLS),
         ti_ref[...].reshape(_D_MLP, _COLS),
         fu_ref[...].reshape(_D_MF, _COLS),
         fi_ref[...].reshape(_D_MF, _COLS)], axis=0)
    p_ref[...] = lax.dot_general(x2, rep_ref[...], (((0,), (0,)), ((), ())),
                                 preferred_element_type=jnp.float32)


def _relayout(tu, ti, fu, fi):
    blk3 = lambda p: pl.BlockSpec((p, 8, _COLS), lambda n: (0, 0, n))
    rep = jnp.eye(_F, 128, dtype=jnp.float32)
    return pl.pallas_call(
        _relayout_body,
        grid=(_GRID_A,),
        in_specs=[blk3(4), blk3(4), blk3(2), blk3(2),
                  pl.BlockSpec((_F, 128), lambda n: (0, 0))],
        out_specs=pl.BlockSpec((_COLS, 128), lambda n: (n, 0)),
        out_shape=jax.ShapeDtypeStruct((_V, 128), jnp.float32),
    )(tu, ti, fu, fi, rep)


def _gather_body(u_hbm, i_hbm, t_p,
                 o_u, o_i,
                 ux, ix, b_u, b_i, sem):
    wid = lax.axis_index("s") * _NC + lax.axis_index("c")
    base = wid * _BPW
    for j in range(_NCH):
        sl = pl.ds(base + j * _CH, _CH)
        pltpu.sync_copy(u_hbm.at[sl], ux.at[j])
        pltpu.sync_copy(i_hbm.at[sl], ix.at[j])
    for j in range(_NCH):
        cps = [
            pltpu.async_copy(t_p.at[ux.at[j]], b_u, sem),
            pltpu.async_copy(t_p.at[ix.at[j]], b_i, sem),
        ]
        for cp in cps:
            cp.wait()
        sl = pl.ds(j * _CH, _CH)
        pltpu.sync_copy(b_u, o_u.at[wid].at[sl])
        pltpu.sync_copy(b_i, o_i.at[wid].at[sl])


@functools.lru_cache(maxsize=None)
def _make_gather():
  return functools.partial(
    pl.kernel,
    mesh=plsc.VectorSubcoreMesh(core_axis_name="c", subcore_axis_name="s"),
    out_type=[
        jax.ShapeDtypeStruct((_NW, _BPW, 128), jnp.float32),
        jax.ShapeDtypeStruct((_NW, _BPW, 128), jnp.float32),
    ],
    scratch_types=[
        pltpu.VMEM((_NCH, _CH), jnp.int32),
        pltpu.VMEM((_NCH, _CH), jnp.int32),
        pltpu.VMEM((_CH, 128), jnp.float32),
        pltpu.VMEM((_CH, 128), jnp.float32),
        pltpu.SemaphoreType.DMA,
    ],
  )(_gather_body)


def _mlp_body(um_ref, im_ref,
              w1u_ref, w1i_ref, b1_ref, w2_ref, b2_ref, w3_ref, b3_ref,
              wom_ref, wof_ref, bo_ref, out_ref):
    pu = um_ref[0]
    pi = im_ref[0]
    xu = pu[:, :_D_MLP]
    xi = pi[:, _D_MLP:2 * _D_MLP]
    mu = pu[:, 2 * _D_MLP:2 * _D_MLP + _D_MF]
    mi = pi[:, 2 * _D_MLP + _D_MF:_F]
    x = jnp.dot(xu, w1u_ref[...], preferred_element_type=jnp.float32)
    x = x + jnp.dot(xi, w1i_ref[...], preferred_element_type=jnp.float32)
    h = jnp.maximum(x + b1_ref[...], 0.0)
    h = jnp.maximum(
        jnp.dot(h, w2_ref[...], preferred_element_type=jnp.float32) + b2_ref[...], 0.0)
    h = jnp.maximum(
        jnp.dot(h, w3_ref[...], preferred_element_type=jnp.float32) + b3_ref[...], 0.0)
    mf = mu * mi
    logit = (jnp.dot(h, wom_ref[...], preferred_element_type=jnp.float32)
             + jnp.dot(mf, wof_ref[...], preferred_element_type=jnp.float32)
             + bo_ref[...])
    out_ref[...] = 1.0 / (1.0 + jnp.exp(-logit))


def _mlp(um, im, w1u, w1i, b1, w2, b2, w3, b3, wom, wof, bo):
    full = lambda n: (0, 0)
    row2 = lambda n: (n, 0)
    blk3 = lambda n: (n, 0, 0)
    return pl.pallas_call(
        _mlp_body,
        grid=(_NW,),
        in_specs=[
            pl.BlockSpec((1, _BPW, 128), blk3),
            pl.BlockSpec((1, _BPW, 128), blk3),
            pl.BlockSpec(w1u.shape, full),
            pl.BlockSpec(w1i.shape, full),
            pl.BlockSpec(b1.shape, full),
            pl.BlockSpec(w2.shape, full),
            pl.BlockSpec(b2.shape, full),
            pl.BlockSpec(w3.shape, full),
            pl.BlockSpec(b3.shape, full),
            pl.BlockSpec(wom.shape, full),
            pl.BlockSpec(wof.shape, full),
            pl.BlockSpec(bo.shape, full),
        ],
        out_specs=pl.BlockSpec((_BPW, 1), row2),
        out_shape=jax.ShapeDtypeStruct((_B, 1), jnp.float32),
    )(um, im, w1u, w1i, b1, w2, b2, w3, b3, wom, wof, bo)


def kernel(u, i, emb_user_mlp, emb_item_mlp, emb_user_mf, emb_item_mf,
           W1, b1, W2, b2, W3, b3, W_out, b_out):
    u = u.astype(jnp.int32)
    i = i.astype(jnp.int32)
    p_cat = _relayout(
        emb_user_mlp.T.reshape(4, 8, _V), emb_item_mlp.T.reshape(4, 8, _V),
        emb_user_mf.T.reshape(2, 8, _V), emb_item_mf.T.reshape(2, 8, _V))
    um, im = _make_gather()(u, i, p_cat)
    y = _mlp(um, im,
             W1[:_D_MLP], W1[_D_MLP:], b1.reshape(1, -1),
             W2, b2.reshape(1, -1), W3, b3.reshape(1, -1),
             W_out[:-_D_MF], W_out[-_D_MF:], b_out.reshape(1, 1))
    return y


# speedup vs baseline: 4.4840x; 1.0346x over previous
import functools

import jax
import jax.numpy as jnp
from jax import lax
from jax.experimental import pallas as pl
from jax.experimental.pallas import tpu as pltpu
from jax.experimental.pallas import tpu_sc as plsc

_B = 16384
_V = 1_000_000
_D_MLP = 32
_D_MF = 16
_F = 2 * _D_MLP + 2 * _D_MF
_NC = 2
_NS = 16
_NW = _NC * _NS
_BPW = _B // _NW
_CH = 128
_NCH = _BPW // _CH

_COLS = 16384
_GRID_A = (_V + _COLS - 1) // _COLS


def _relayout_body(tu_ref, ti_ref, fu_ref, fi_ref, rep_ref, p_ref):
    x2 = jnp.concatenate(
        [tu_ref[...].reshape(_D_MLP, _COLS),
         ti_ref[...].reshape(_D_MLP, _COLS),
         fu_ref[...].reshape(_D_MF, _COLS),
         fi_ref[...].reshape(_D_MF, _COLS)], axis=0)
    p_ref[...] = lax.dot_general(x2, rep_ref[...], (((0,), (0,)), ((), ())),
                                 preferred_element_type=jnp.float32)


def _relayout(tu, ti, fu, fi):
    blk3 = lambda p: pl.BlockSpec((p, 8, _COLS), lambda n: (0, 0, n))
    rep = jnp.eye(_F, 128, dtype=jnp.float32)
    return pl.pallas_call(
        _relayout_body,
        grid=(_GRID_A,),
        in_specs=[blk3(4), blk3(4), blk3(2), blk3(2),
                  pl.BlockSpec((_F, 128), lambda n: (0, 0))],
        out_specs=pl.BlockSpec((_COLS, 128), lambda n: (n, 0)),
        out_shape=jax.ShapeDtypeStruct((_V, 128), jnp.float32),
    )(tu, ti, fu, fi, rep)


def _gather_body(u_hbm, i_hbm, t_p,
                 o_u, o_i,
                 ux, ix, b_u, b_i, sem):
    wid = lax.axis_index("s") * _NC + lax.axis_index("c")
    base = wid * _BPW
    for j in range(_NCH):
        sl = pl.ds(base + j * _CH, _CH)
        pltpu.sync_copy(u_hbm.at[sl], ux.at[j])
        pltpu.sync_copy(i_hbm.at[sl], ix.at[j])
    for j in range(_NCH):
        cps = [
            pltpu.async_copy(t_p.at[ux.at[j]], b_u, sem),
            pltpu.async_copy(t_p.at[ix.at[j]], b_i, sem),
        ]
        for cp in cps:
            cp.wait()
        sl = pl.ds(j * _CH, _CH)
        pltpu.sync_copy(b_u, o_u.at[wid].at[sl])
        pltpu.sync_copy(b_i, o_i.at[wid].at[sl])


@functools.lru_cache(maxsize=None)
def _make_gather():
  return functools.partial(
    pl.kernel,
    mesh=plsc.VectorSubcoreMesh(core_axis_name="c", subcore_axis_name="s"),
    out_type=[
        jax.ShapeDtypeStruct((_NW, _BPW, 128), jnp.float32),
        jax.ShapeDtypeStruct((_NW, _BPW, 128), jnp.float32),
    ],
    scratch_types=[
        pltpu.VMEM((_NCH, _CH), jnp.int32),
        pltpu.VMEM((_NCH, _CH), jnp.int32),
        pltpu.VMEM((_CH, 128), jnp.float32),
        pltpu.VMEM((_CH, 128), jnp.float32),
        pltpu.SemaphoreType.DMA,
    ],
  )(_gather_body)


def _mlp_body(um_ref, im_ref,
              w1u_ref, w1i_ref, b1_ref, w2_ref, b2_ref, w3_ref, b3_ref,
              wom_ref, wof_ref, bo_ref, out_ref):
    pu = um_ref[0]
    pi = im_ref[0]
    xu = pu[:, :_D_MLP]
    xi = pi[:, _D_MLP:2 * _D_MLP]
    mu = pu[:, 2 * _D_MLP:2 * _D_MLP + _D_MF]
    mi = pi[:, 2 * _D_MLP + _D_MF:_F]
    x = jnp.dot(xu, w1u_ref[...], preferred_element_type=jnp.float32)
    x = x + jnp.dot(xi, w1i_ref[...], preferred_element_type=jnp.float32)
    h = jnp.maximum(x + b1_ref[...], 0.0)
    h = jnp.maximum(
        jnp.dot(h, w2_ref[...], preferred_element_type=jnp.float32) + b2_ref[...], 0.0)
    h = jnp.maximum(
        jnp.dot(h, w3_ref[...], preferred_element_type=jnp.float32) + b3_ref[...], 0.0)
    mf = mu * mi
    logit = (jnp.dot(h, wom_ref[...], preferred_element_type=jnp.float32)
             + jnp.dot(mf, wof_ref[...], preferred_element_type=jnp.float32)
             + bo_ref[...])
    out_ref[...] = 1.0 / (1.0 + jnp.exp(-logit))


def _mlp(um, im, w1u, w1i, b1, w2, b2, w3, b3, wom, wof, bo):
    full = lambda n: (0, 0)
    row2 = lambda n: (n, 0)
    blk3 = lambda n: (n, 0, 0)
    return pl.pallas_call(
        _mlp_body,
        grid=(_NW,),
        in_specs=[
            pl.BlockSpec((1, _BPW, 128), blk3),
            pl.BlockSpec((1, _BPW, 128), blk3),
            pl.BlockSpec(w1u.shape, full),
            pl.BlockSpec(w1i.shape, full),
            pl.BlockSpec(b1.shape, full),
            pl.BlockSpec(w2.shape, full),
            pl.BlockSpec(b2.shape, full),
            pl.BlockSpec(w3.shape, full),
            pl.BlockSpec(b3.shape, full),
            pl.BlockSpec(wom.shape, full),
            pl.BlockSpec(wof.shape, full),
            pl.BlockSpec(bo.shape, full),
        ],
        out_specs=pl.BlockSpec((_BPW, 1), row2),
        out_shape=jax.ShapeDtypeStruct((_B, 1), jnp.float32),
    )(um, im, w1u, w1i, b1, w2, b2, w3, b3, wom, wof, bo)


def kernel(u, i, emb_user_mlp, emb_item_mlp, emb_user_mf, emb_item_mf,
           W1, b1, W2, b2, W3, b3, W_out, b_out):
    u = u.astype(jnp.int32)
    i = i.astype(jnp.int32)
    p_cat = _relayout(
        emb_user_mlp.T.reshape(4, 8, _V), emb_item_mlp.T.reshape(4, 8, _V),
        emb_user_mf.T.reshape(2, 8, _V), emb_item_mf.T.reshape(2, 8, _V))
    um, im = _make_gather()(u, i, p_cat)
    y = _mlp(um, im,
             W1[:_D_MLP], W1[_D_MLP:], b1.reshape(1, -1),
             W2, b2.reshape(1, -1), W3, b3.reshape(1, -1),
             W_out[:-_D_MF], W_out[-_D_MF:], b_out.reshape(1, 1))
    return y
